# baseline (speedup 1.0000x reference)
import jax
import jax.numpy as jnp
from jax.experimental import pallas as pl
from jax.experimental.pallas import tpu as pltpu


def _outer(idx_ref, wgt_ref, x_ref, gup_hbm, down_hbm, out_ref):
    num_experts = gup_hbm.shape[0]
    hidden = gup_hbm.shape[1]
    interm = down_hbm.shape[1]
    out_ref[...] = jnp.zeros_like(out_ref)
    x = x_ref[...].astype(jnp.bfloat16)
    idx = idx_ref[...]
    wgt = wgt_ref[...]

    def body(gate_blk, up_blk, down_blk):
        e = pl.program_id(0)
        gate = jnp.dot(x, gate_blk[0].astype(jnp.bfloat16),
                       preferred_element_type=jnp.float32)
        up = jnp.dot(x, up_blk[0].astype(jnp.bfloat16),
                     preferred_element_type=jnp.float32)
        act = gate * jax.nn.sigmoid(gate) * up
        oe = jnp.dot(act.astype(jnp.bfloat16), down_blk[0].astype(jnp.bfloat16),
                     preferred_element_type=jnp.float32)
        w = jnp.sum(jnp.where(idx == e, wgt, 0.0), axis=0)
        out_ref[...] += w[:, None] * oe

    pltpu.emit_pipeline(
        body,
        grid=(num_experts,),
        in_specs=[
            pl.BlockSpec((1, hidden, interm), lambda e: (e, 0, 0),
                         pipeline_mode=pl.Buffered(buffer_count=4)),
            pl.BlockSpec((1, hidden, interm), lambda e: (e, 0, 1),
                         pipeline_mode=pl.Buffered(buffer_count=4)),
            pl.BlockSpec((1, interm, hidden), lambda e: (e, 0, 0),
                         pipeline_mode=pl.Buffered(buffer_count=4)),
        ],
    )(gup_hbm, gup_hbm, down_hbm)


def kernel(hidden_states, gate_up_proj, down_proj, expert_indices, expert_weights):
    num_tokens, hidden = hidden_states.shape
    idx_t = expert_indices.astype(jnp.int32).T
    wgt_t = expert_weights.T

    return pl.pallas_call(
        _outer,
        in_specs=[
            pl.BlockSpec(memory_space=pltpu.MemorySpace.VMEM),
            pl.BlockSpec(memory_space=pltpu.MemorySpace.VMEM),
            pl.BlockSpec(memory_space=pltpu.MemorySpace.VMEM),
            pl.BlockSpec(memory_space=pltpu.MemorySpace.HBM),
            pl.BlockSpec(memory_space=pltpu.MemorySpace.HBM),
        ],
        out_specs=pl.BlockSpec(memory_space=pltpu.MemorySpace.VMEM),
        out_shape=jax.ShapeDtypeStruct((num_tokens, hidden), jnp.float32),
    )(idx_t, wgt_t, hidden_states, gate_up_proj, down_proj)


# speedup vs baseline: 1.0062x; 1.0062x over previous
import jax
import jax.numpy as jnp
from jax.experimental import pallas as pl
from jax.experimental.pallas import tpu as pltpu


def _outer(idx_ref, wgt_ref, x_ref, gup_hbm, down_hbm, out_ref):
    num_experts = gup_hbm.shape[0]
    hidden = gup_hbm.shape[1]
    interm = down_hbm.shape[1]
    out_ref[...] = jnp.zeros_like(out_ref)
    x = x_ref[...].astype(jnp.bfloat16)
    idx = idx_ref[...]
    wgt = wgt_ref[...]

    def body(gate_blk, up_blk, down_blk):
        e = pl.program_id(0)
        gate = jnp.dot(x, gate_blk[0].astype(jnp.bfloat16),
                       preferred_element_type=jnp.float32)
        up = jnp.dot(x, up_blk[0].astype(jnp.bfloat16),
                     preferred_element_type=jnp.float32)
        w = jnp.sum(jnp.where(idx == e, wgt, 0.0), axis=0)
        act = (gate * jax.nn.sigmoid(gate) * up) * w[:, None]
        oe = jnp.dot(act.astype(jnp.bfloat16), down_blk[0].astype(jnp.bfloat16),
                     preferred_element_type=jnp.float32)
        out_ref[...] += oe

    pltpu.emit_pipeline(
        body,
        grid=(num_experts,),
        in_specs=[
            pl.BlockSpec((1, hidden, interm), lambda e: (e, 0, 0),
                         pipeline_mode=pl.Buffered(buffer_count=4)),
            pl.BlockSpec((1, hidden, interm), lambda e: (e, 0, 1),
                         pipeline_mode=pl.Buffered(buffer_count=4)),
            pl.BlockSpec((1, interm, hidden), lambda e: (e, 0, 0),
                         pipeline_mode=pl.Buffered(buffer_count=4)),
        ],
    )(gup_hbm, gup_hbm, down_hbm)


def kernel(hidden_states, gate_up_proj, down_proj, expert_indices, expert_weights):
    num_tokens, hidden = hidden_states.shape
    idx_t = expert_indices.astype(jnp.int32).T
    wgt_t = expert_weights.T

    return pl.pallas_call(
        _outer,
        in_specs=[
            pl.BlockSpec(memory_space=pltpu.MemorySpace.VMEM),
            pl.BlockSpec(memory_space=pltpu.MemorySpace.VMEM),
            pl.BlockSpec(memory_space=pltpu.MemorySpace.VMEM),
            pl.BlockSpec(memory_space=pltpu.MemorySpace.HBM),
            pl.BlockSpec(memory_space=pltpu.MemorySpace.HBM),
        ],
        out_specs=pl.BlockSpec(memory_space=pltpu.MemorySpace.VMEM),
        out_shape=jax.ShapeDtypeStruct((num_tokens, hidden), jnp.float32),
    )(idx_t, wgt_t, hidden_states, gate_up_proj, down_proj)
